# baseline (speedup 1.0000x reference)
import functools

import jax
import jax.numpy as jnp
from jax import lax
from jax.experimental import pallas as pl
from jax.experimental.pallas import tpu as pltpu
from jax.experimental.pallas import tpu_sc as plsc

NUM_USERS = 100000
NUM_ITEMS = 100000
EMB = 64
T = 3
BATCH = 16384
LAMB = 0.5

NC = 2
NS = 16
NW = NC * NS
BPW = BATCH // NW
CHUNK = 128
NCHUNK = BPW // CHUNK

_SC_MESH = plsc.VectorSubcoreMesh(core_axis_name="c", subcore_axis_name="s")


@functools.partial(
    pl.kernel,
    out_type=jax.ShapeDtypeStruct((8, BATCH, EMB), jnp.float32),
    mesh=_SC_MESH,
    scratch_types=[
        pltpu.VMEM((NCHUNK, CHUNK), jnp.int32),
        pltpu.VMEM((BPW, EMB), jnp.float32),
        pltpu.SemaphoreType.DMA,
    ],
    compiler_params=pltpu.CompilerParams(use_tc_tiling_on_sc=False),
)
def _sc_gather(ul, il, um, ss, idx, out, idx_v, rows_v, sem):
    wid = lax.axis_index("s") * NC + lax.axis_index("c")
    tables = (ul, il, um, um, um, ss, ss, ss)
    for g in range(8):
        pltpu.sync_copy(idx.at[g, wid], idx_v)
        copies = [
            pltpu.async_copy(
                tables[g].at[idx_v.at[j]],
                rows_v.at[pl.ds(j * CHUNK, CHUNK)],
                sem,
            )
            for j in range(NCHUNK)
        ]
        for c in copies:
            c.wait()
        pltpu.sync_copy(rows_v, out.at[g, pl.ds(wid * BPW, BPW)])


TN = 4096


def _transpose_body(ult_ref, ilt_ref, umt_ref, sst_ref, out_ref):
    out_ref[0] = jnp.transpose(ult_ref[...], (1, 0))
    out_ref[1] = jnp.transpose(ilt_ref[...], (1, 0))
    for t in range(T):
        out_ref[2 + t] = jnp.transpose(umt_ref[t], (1, 0))
        out_ref[5 + t] = jnp.transpose(sst_ref[t], (1, 0))


def _transpose_all(ult, ilt, umt, sst):
    nb = (NUM_USERS + TN - 1) // TN
    return pl.pallas_call(
        _transpose_body,
        grid=(nb,),
        in_specs=[
            pl.BlockSpec((EMB, TN), lambda i: (0, i)),
            pl.BlockSpec((EMB, TN), lambda i: (0, i)),
            pl.BlockSpec((T, EMB, TN), lambda i: (0, 0, i)),
            pl.BlockSpec((T, EMB, TN), lambda i: (0, 0, i)),
        ],
        out_specs=pl.BlockSpec((8, TN, EMB), lambda i: (0, i, 0)),
        out_shape=jax.ShapeDtypeStruct((8, NUM_USERS, EMB), jnp.float32),
    )(ult, ilt, umt, sst)


BLK = 2048


def _tc_body(rows_ref, m_ref, out_ref):
    u = rows_ref[0]
    i = rows_ref[1]
    acc = LAMB * jnp.sum(u * i, axis=-1, keepdims=True)
    w = (1.0 - LAMB) / T
    for t in range(T):
        p = rows_ref[2 + t]
        s = rows_ref[5 + t]
        proj = lax.dot_general(
            p, m_ref[t], (((1,), (0,)), ((), ())),
            precision=lax.Precision.HIGHEST,
            preferred_element_type=jnp.float32,
        )
        acc = acc + w * jnp.sum(proj * s, axis=-1, keepdims=True)
    out_ref[...] = acc


def kernel(user_idx, item_idx, user_latent, item_latent, s_item_list,
           user_mean_emb, M_t):
    ui = user_idx.astype(jnp.int32)
    ii = item_idx.astype(jnp.int32)
    ult = jnp.swapaxes(user_latent, 0, 1)
    ilt = jnp.swapaxes(item_latent, 0, 1)
    umt = jnp.transpose(user_mean_emb, (0, 2, 1))
    sst = jnp.transpose(s_item_list, (0, 2, 1))
    planes = _transpose_all(ult, ilt, umt, sst)
    ul = planes[0]
    il = planes[1]
    um = planes[2:5].reshape(T * NUM_USERS, EMB)
    ss = planes[5:8].reshape(T * NUM_ITEMS, EMB)
    offs_u = jnp.arange(T, dtype=jnp.int32)[:, None] * NUM_USERS
    offs_i = jnp.arange(T, dtype=jnp.int32)[:, None] * NUM_ITEMS
    idx_all = jnp.concatenate(
        [ui[None], ii[None], ui[None] + offs_u, ii[None] + offs_i], axis=0
    ).reshape(8, NW, NCHUNK, CHUNK)

    rows = _sc_gather(ul, il, um, ss, idx_all)

    score2 = pl.pallas_call(
        _tc_body,
        grid=(BATCH // BLK,),
        in_specs=[
            pl.BlockSpec((8, BLK, EMB), lambda i: (0, i, 0)),
            pl.BlockSpec((T, EMB, EMB), lambda i: (0, 0, 0)),
        ],
        out_specs=pl.BlockSpec((BLK, 1), lambda i: (i, 0)),
        out_shape=jax.ShapeDtypeStruct((BATCH, 1), jnp.float32),
    )(rows, M_t)
    return score2[:, 0]


# speedup vs baseline: 1.4115x; 1.4115x over previous
import functools

import jax
import jax.numpy as jnp
from jax import lax
from jax.experimental import pallas as pl
from jax.experimental.pallas import tpu as pltpu
from jax.experimental.pallas import tpu_sc as plsc

NUM_USERS = 100000
NUM_ITEMS = 100000
EMB = 64
T = 3
BATCH = 16384
LAMB = 0.5

NC = 2
NS = 16
NW = NC * NS
BPW = BATCH // NW
CHUNK = 128
NCHUNK = BPW // CHUNK

_SC_MESH = plsc.VectorSubcoreMesh(core_axis_name="c", subcore_axis_name="s")


@functools.partial(
    pl.kernel,
    out_type=jax.ShapeDtypeStruct((8, BATCH, EMB), jnp.float32),
    mesh=_SC_MESH,
    scratch_types=[
        pltpu.VMEM((NCHUNK, CHUNK), jnp.int32),
        pltpu.VMEM((BPW, EMB), jnp.float32),
        pltpu.SemaphoreType.DMA,
    ],
    compiler_params=pltpu.CompilerParams(use_tc_tiling_on_sc=False),
)
def _sc_gather(planes, idx, out, idx_v, rows_v, sem):
    wid = lax.axis_index("s") * NC + lax.axis_index("c")
    for g in range(8):
        pltpu.sync_copy(idx.at[g, wid], idx_v)
        copies = [
            pltpu.async_copy(
                planes.at[idx_v.at[j]],
                rows_v.at[pl.ds(j * CHUNK, CHUNK)],
                sem,
            )
            for j in range(NCHUNK)
        ]
        for c in copies:
            c.wait()
        pltpu.sync_copy(rows_v, out.at[g, pl.ds(wid * BPW, BPW)])


TN = 4096


def _transpose_body(ult_ref, ilt_ref, umt_ref, sst_ref, out_ref):
    out_ref[0] = jnp.transpose(ult_ref[...], (1, 0))
    out_ref[1] = jnp.transpose(ilt_ref[...], (1, 0))
    for t in range(T):
        out_ref[2 + t] = jnp.transpose(umt_ref[t], (1, 0))
        out_ref[5 + t] = jnp.transpose(sst_ref[t], (1, 0))


def _transpose_all(ult, ilt, umt, sst):
    nb = (NUM_USERS + TN - 1) // TN
    return pl.pallas_call(
        _transpose_body,
        grid=(nb,),
        in_specs=[
            pl.BlockSpec((EMB, TN), lambda i: (0, i)),
            pl.BlockSpec((EMB, TN), lambda i: (0, i)),
            pl.BlockSpec((T, EMB, TN), lambda i: (0, 0, i)),
            pl.BlockSpec((T, EMB, TN), lambda i: (0, 0, i)),
        ],
        out_specs=pl.BlockSpec((8, TN, EMB), lambda i: (0, i, 0)),
        out_shape=jax.ShapeDtypeStruct((8, NUM_USERS, EMB), jnp.float32),
    )(ult, ilt, umt, sst)


BLK = 2048


def _tc_body(rows_ref, m_ref, out_ref):
    u = rows_ref[0]
    i = rows_ref[1]
    acc = LAMB * jnp.sum(u * i, axis=-1, keepdims=True)
    w = (1.0 - LAMB) / T
    for t in range(T):
        p = rows_ref[2 + t]
        s = rows_ref[5 + t]
        proj = lax.dot_general(
            p, m_ref[t], (((1,), (0,)), ((), ())),
            precision=lax.Precision.HIGHEST,
            preferred_element_type=jnp.float32,
        )
        acc = acc + w * jnp.sum(proj * s, axis=-1, keepdims=True)
    out_ref[...] = acc


def kernel(user_idx, item_idx, user_latent, item_latent, s_item_list,
           user_mean_emb, M_t):
    ui = user_idx.astype(jnp.int32)
    ii = item_idx.astype(jnp.int32)
    ult = jnp.swapaxes(user_latent, 0, 1)
    ilt = jnp.swapaxes(item_latent, 0, 1)
    umt = jnp.transpose(user_mean_emb, (0, 2, 1))
    sst = jnp.transpose(s_item_list, (0, 2, 1))
    planes = _transpose_all(ult, ilt, umt, sst)
    planes_flat = planes.reshape(8 * NUM_USERS, EMB)
    offs_u = (jnp.arange(2, 5, dtype=jnp.int32)[:, None] * NUM_USERS)
    offs_i = (jnp.arange(5, 8, dtype=jnp.int32)[:, None] * NUM_ITEMS)
    idx_all = jnp.concatenate(
        [ui[None], ii[None] + NUM_ITEMS, ui[None] + offs_u,
         ii[None] + offs_i], axis=0
    ).reshape(8, NW, NCHUNK, CHUNK)

    rows = _sc_gather(planes_flat, idx_all)

    score2 = pl.pallas_call(
        _tc_body,
        grid=(BATCH // BLK,),
        in_specs=[
            pl.BlockSpec((8, BLK, EMB), lambda i: (0, i, 0)),
            pl.BlockSpec((T, EMB, EMB), lambda i: (0, 0, 0)),
        ],
        out_specs=pl.BlockSpec((BLK, 1), lambda i: (i, 0)),
        out_shape=jax.ShapeDtypeStruct((BATCH, 1), jnp.float32),
    )(rows, M_t)
    return score2[:, 0]


# speedup vs baseline: 3.6043x; 2.5536x over previous
import functools

import jax
import jax.numpy as jnp
from jax import lax
from jax.experimental import pallas as pl
from jax.experimental.pallas import tpu as pltpu
from jax.experimental.pallas import tpu_sc as plsc

NUM_USERS = 100000
NUM_ITEMS = 100000
EMB = 64
T = 3
BATCH = 16384
LAMB = 0.5

NC = 2
NS = 16
NW = NC * NS
BPW = BATCH // NW
CHUNK = 128
NCHUNK = BPW // CHUNK

_SC_MESH = plsc.VectorSubcoreMesh(core_axis_name="c", subcore_axis_name="s")


@functools.partial(
    pl.kernel,
    out_type=jax.ShapeDtypeStruct((4, BATCH, 2 * EMB), jnp.float32),
    mesh=_SC_MESH,
    scratch_types=[
        pltpu.VMEM((NCHUNK, CHUNK), jnp.int32),
        pltpu.VMEM((BPW, 2 * EMB), jnp.float32),
        pltpu.SemaphoreType.DMA,
    ],
    compiler_params=pltpu.CompilerParams(use_tc_tiling_on_sc=False),
)
def _sc_gather(planes, idx, out, idx_v, rows_v, sem):
    wid = lax.axis_index("s") * NC + lax.axis_index("c")
    for g in range(4):
        pltpu.sync_copy(idx.at[g, wid], idx_v)
        copies = [
            pltpu.async_copy(
                planes.at[idx_v.at[j]],
                rows_v.at[pl.ds(j * CHUNK, CHUNK)],
                sem,
            )
            for j in range(NCHUNK)
        ]
        for c in copies:
            c.wait()
        pltpu.sync_copy(rows_v, out.at[g, pl.ds(wid * BPW, BPW)])


TN = 4096


def _transpose_body(ult_ref, ilt_ref, umt_ref, sst_ref, out_ref):
    tt = lambda x: jnp.transpose(x, (1, 0))
    out_ref[0] = jnp.concatenate([tt(ult_ref[...]), tt(umt_ref[0])], axis=1)
    out_ref[1] = jnp.concatenate([tt(umt_ref[1]), tt(umt_ref[2])], axis=1)
    out_ref[2] = jnp.concatenate([tt(ilt_ref[...]), tt(sst_ref[0])], axis=1)
    out_ref[3] = jnp.concatenate([tt(sst_ref[1]), tt(sst_ref[2])], axis=1)


def _transpose_all(ult, ilt, umt, sst):
    nb = (NUM_USERS + TN - 1) // TN
    return pl.pallas_call(
        _transpose_body,
        grid=(nb,),
        in_specs=[
            pl.BlockSpec((EMB, TN), lambda i: (0, i)),
            pl.BlockSpec((EMB, TN), lambda i: (0, i)),
            pl.BlockSpec((T, EMB, TN), lambda i: (0, 0, i)),
            pl.BlockSpec((T, EMB, TN), lambda i: (0, 0, i)),
        ],
        out_specs=pl.BlockSpec((4, TN, 2 * EMB), lambda i: (0, i, 0)),
        out_shape=jax.ShapeDtypeStruct((4, NUM_USERS, 2 * EMB), jnp.float32),
    )(ult, ilt, umt, sst)


BLK = 2048


def _tc_body(rows_ref, m_ref, out_ref):
    r0 = rows_ref[0]
    r1 = rows_ref[1]
    r2 = rows_ref[2]
    r3 = rows_ref[3]
    u = r0[:, :EMB]
    i = r2[:, :EMB]
    p = (r0[:, EMB:], r1[:, :EMB], r1[:, EMB:])
    s = (r2[:, EMB:], r3[:, :EMB], r3[:, EMB:])
    acc = LAMB * jnp.sum(u * i, axis=-1, keepdims=True)
    w = (1.0 - LAMB) / T
    for t in range(T):
        proj = lax.dot_general(
            p[t], m_ref[t], (((1,), (0,)), ((), ())),
            precision=lax.Precision.HIGHEST,
            preferred_element_type=jnp.float32,
        )
        acc = acc + w * jnp.sum(proj * s[t], axis=-1, keepdims=True)
    out_ref[...] = acc


def kernel(user_idx, item_idx, user_latent, item_latent, s_item_list,
           user_mean_emb, M_t):
    ui = user_idx.astype(jnp.int32)
    ii = item_idx.astype(jnp.int32)
    ult = jnp.swapaxes(user_latent, 0, 1)
    ilt = jnp.swapaxes(item_latent, 0, 1)
    umt = jnp.transpose(user_mean_emb, (0, 2, 1))
    sst = jnp.transpose(s_item_list, (0, 2, 1))
    planes = _transpose_all(ult, ilt, umt, sst)
    planes_flat = planes.reshape(4 * NUM_USERS, 2 * EMB)
    n = jnp.int32(NUM_USERS)
    idx_all = jnp.concatenate(
        [ui[None], ui[None] + n, ii[None] + 2 * n, ii[None] + 3 * n],
        axis=0,
    ).reshape(4, NW, NCHUNK, CHUNK)

    rows = _sc_gather(planes_flat, idx_all)

    score2 = pl.pallas_call(
        _tc_body,
        grid=(BATCH // BLK,),
        in_specs=[
            pl.BlockSpec((4, BLK, 2 * EMB), lambda i: (0, i, 0)),
            pl.BlockSpec((T, EMB, EMB), lambda i: (0, 0, 0)),
        ],
        out_specs=pl.BlockSpec((BLK, 1), lambda i: (i, 0)),
        out_shape=jax.ShapeDtypeStruct((BATCH, 1), jnp.float32),
    )(rows, M_t)
    return score2[:, 0]
